# baseline (speedup 1.0000x reference)
import functools

import jax
import jax.numpy as jnp
from jax import lax
from jax.experimental import pallas as pl
from jax.experimental.pallas import tpu as pltpu
from jax.experimental.pallas import tpu_sc as plsc

NC = 2
NS = 16
L = 16
NW = NC * NS
BPW = 160
NSTREAM = BPW * 4 // 128


def _make_gather(k, n):
    tail = k - (NW - 1) * BPW
    assert 0 < tail <= BPW and tail % 8 == 0 and (BPW * 4) % 128 == 0
    tail_g = -(-tail // L)
    tg16 = tail_g * L
    tbase = 3 * k
    mesh = plsc.VectorSubcoreMesh(
        core_axis_name="c", subcore_axis_name="s", num_cores=NC
    )

    @functools.partial(
        pl.kernel,
        mesh=mesh,
        out_type=jax.ShapeDtypeStruct((4 * k,), jnp.int32),
        compiler_params=pltpu.CompilerParams(
            needs_layout_passes=False, use_tc_tiling_on_sc=False,
            disable_bounds_checks=True, disable_semaphore_checks=True,
        ),
        scratch_types=[
            pltpu.VMEM((BPW,), jnp.int32),
            pltpu.VMEM((BPW * 4,), jnp.int32),
            pltpu.VMEM((BPW * 4,), jnp.int32),
            pltpu.SemaphoreType.DMA,
        ],
    )
    def gather(packed_hbm, out_hbm, bidx, widx, vals, sem):
        wid = lax.axis_index("s") * NC + lax.axis_index("c")
        base = wid * BPW

        @pl.when(wid < NW - 1)
        def _full():
            pltpu.sync_copy(packed_hbm.at[pl.ds(2 * k + base, BPW)], bidx)
            for g in range(BPW // L):
                b16 = bidx[pl.ds(g * L, L)]
                for c in range(4):
                    widx[pl.ds(c * BPW + g * L, L)] = b16 + (tbase + c * n)
            cps = [
                pltpu.async_copy(
                    packed_hbm.at[widx.at[pl.ds(s * 128, 128)]],
                    vals.at[pl.ds(s * 128, 128)], sem)
                for s in range(NSTREAM)
            ]
            for cp in cps:
                cp.wait()
            for c in range(4):
                pltpu.sync_copy(vals.at[pl.ds(c * BPW, BPW)],
                                out_hbm.at[pl.ds(c * k + base, BPW)])

        @pl.when(wid == NW - 1)
        def _short_tail():
            pltpu.sync_copy(packed_hbm.at[pl.ds(2 * k + base, tail)],
                            bidx.at[pl.ds(0, tail)])
            for g in range(tail_g):
                b16 = bidx[pl.ds(g * L, L)]
                b16 = jnp.minimum(jnp.maximum(b16, 0), n - 1)
                for c in range(4):
                    widx[pl.ds(c * tg16 + g * L, L)] = b16 + (tbase + c * n)
            cps = [
                pltpu.async_copy(
                    packed_hbm.at[widx.at[pl.ds(s * 64, 64)]],
                    vals.at[pl.ds(s * 64, 64)], sem)
                for s in range(4 * tg16 // 64)
            ]
            for cp in cps:
                cp.wait()
            for c in range(4):
                pltpu.sync_copy(vals.at[pl.ds(c * tg16, tail)],
                                out_hbm.at[pl.ds(c * k + base, tail)])

    return gather


def kernel(selected_indices, xyxy_boxes):
    k = selected_indices.shape[0]
    n = xyxy_boxes.shape[1]
    sel_t = selected_indices.astype(jnp.int32).T
    table_i = lax.bitcast_convert_type(xyxy_boxes[0].T, jnp.int32)
    packed = jnp.concatenate([sel_t.reshape(-1), table_i.reshape(-1)])
    out = _make_gather(k, n)(packed)
    return lax.bitcast_convert_type(out.reshape(4, k).T, jnp.float32)


# speedup vs baseline: 1.0492x; 1.0492x over previous
import functools

import jax
import jax.numpy as jnp
from jax import lax
from jax.experimental import pallas as pl
from jax.experimental.pallas import tpu as pltpu
from jax.experimental.pallas import tpu_sc as plsc

NC = 1
NS = 16
L = 16
NW = NC * NS
BPW = 320
NSTREAM = BPW * 4 // 128


def _make_gather(k, n):
    tail = k - (NW - 1) * BPW
    assert 0 < tail <= BPW and tail % 8 == 0 and (BPW * 4) % 128 == 0
    tail_g = -(-tail // L)
    tg16 = tail_g * L
    tbase = 3 * k
    mesh = plsc.VectorSubcoreMesh(
        core_axis_name="c", subcore_axis_name="s", num_cores=NC
    )

    @functools.partial(
        pl.kernel,
        mesh=mesh,
        out_type=jax.ShapeDtypeStruct((4 * k,), jnp.int32),
        compiler_params=pltpu.CompilerParams(
            needs_layout_passes=False, use_tc_tiling_on_sc=False,
            disable_bounds_checks=True, disable_semaphore_checks=True,
        ),
        scratch_types=[
            pltpu.VMEM((BPW,), jnp.int32),
            pltpu.VMEM((BPW * 4,), jnp.int32),
            pltpu.VMEM((BPW * 4,), jnp.int32),
            [pltpu.SemaphoreType.DMA] * 4,
        ],
    )
    def gather(packed_hbm, out_hbm, bidx, widx, vals, sems):
        wid = lax.axis_index("s") * NC + lax.axis_index("c")
        base = wid * BPW

        @pl.when(wid < NW - 1)
        def _full():
            pltpu.sync_copy(packed_hbm.at[pl.ds(2 * k + base, BPW)], bidx)
            for g in range(BPW // L):
                b16 = bidx[pl.ds(g * L, L)]
                for c in range(4):
                    widx[pl.ds(c * BPW + g * L, L)] = b16 + (tbase + c * n)
            cps = [
                [
                    pltpu.async_copy(
                        packed_hbm.at[widx.at[pl.ds(c * BPW + o, sz)]],
                        vals.at[pl.ds(c * BPW + o, sz)], sems[c])
                    for o, sz in ((0, 128), (128, 128), (256, BPW - 256))
                ]
                for c in range(4)
            ]
            for c in range(4):
                for cp in cps[c]:
                    cp.wait()
                pltpu.sync_copy(vals.at[pl.ds(c * BPW, BPW)],
                                out_hbm.at[pl.ds(c * k + base, BPW)])

        @pl.when(wid == NW - 1)
        def _short_tail():
            pltpu.sync_copy(packed_hbm.at[pl.ds(2 * k + base, tail)],
                            bidx.at[pl.ds(0, tail)])
            for g in range(tail_g):
                b16 = bidx[pl.ds(g * L, L)]
                b16 = jnp.minimum(jnp.maximum(b16, 0), n - 1)
                for c in range(4):
                    widx[pl.ds(c * tg16 + g * L, L)] = b16 + (tbase + c * n)
            cps = [
                pltpu.async_copy(
                    packed_hbm.at[widx.at[pl.ds(s * 64, 64)]],
                    vals.at[pl.ds(s * 64, 64)], sems[0])
                for s in range(4 * tg16 // 64)
            ]
            for cp in cps:
                cp.wait()
            for c in range(4):
                pltpu.sync_copy(vals.at[pl.ds(c * tg16, tail)],
                                out_hbm.at[pl.ds(c * k + base, tail)])

    return gather


def kernel(selected_indices, xyxy_boxes):
    k = selected_indices.shape[0]
    n = xyxy_boxes.shape[1]
    sel_t = selected_indices.astype(jnp.int32).T
    table_i = lax.bitcast_convert_type(xyxy_boxes[0].T, jnp.int32)
    packed = jnp.concatenate([sel_t.reshape(-1), table_i.reshape(-1)])
    out = _make_gather(k, n)(packed)
    return lax.bitcast_convert_type(out.reshape(4, k).T, jnp.float32)
